# Initial kernel scaffold; baseline (speedup 1.0000x reference)
import jax
import jax.numpy as jnp
from jax.experimental import pallas as pl


def kernel(x, edge_index, W1, b1, W2, b2):
    raise NotImplementedError("write your pallas kernel here")


# speedup vs baseline: 12.2714x; 12.2714x over previous
import functools

import jax
import jax.numpy as jnp
from jax import lax
from jax.experimental import pallas as pl
from jax.experimental.pallas import tpu as pltpu
from jax.experimental.pallas import tpu_sc as plsc

N = 10000
E = 160000
D_IN = 500
H = 16
C = 3

N_PAD = 10240
D_PAD = 512
NC = 2
NS = 16
NW = NC * NS
CHUNK = 128
NCHUNK = 40
E_PAD = NW * CHUNK * NCHUNK
EPW = CHUNK * NCHUNK
RPW = N_PAD // NS
BLK = 512
GRID = N_PAD // BLK

@functools.cache
def _sc_kernels():
    mesh = plsc.VectorSubcoreMesh(core_axis_name="c", subcore_axis_name="s")
    params = pltpu.CompilerParams(use_tc_tiling_on_sc=False)

    @functools.partial(
        pl.kernel,
        mesh=mesh,
        compiler_params=params,
        out_type=jax.ShapeDtypeStruct((NC, N_PAD, H), jnp.float32),
        scratch_types=[
            pltpu.VMEM_SHARED((N_PAD, H), jnp.float32),
            pltpu.VMEM((CHUNK,), jnp.int32),
            pltpu.VMEM((CHUNK, H), jnp.float32),
        ],
    )
    def _sc_degree(ones_hbm, dst_hbm, out_hbm, acc_sh, dst_v, ones_v):
        c = lax.axis_index("c")
        s = lax.axis_index("s")
        w = c * NS + s
        r0 = s * RPW
        pltpu.sync_copy(ones_hbm.at[pl.ds(r0, RPW)], acc_sh.at[pl.ds(r0, RPW)])
        pltpu.sync_copy(ones_hbm.at[pl.ds(0, CHUNK)], ones_v)
        plsc.subcore_barrier()
        base = w * EPW

        def body(j, carry):
            b = base + j * CHUNK
            pltpu.sync_copy(dst_hbm.at[pl.ds(b, CHUNK)], dst_v)
            pltpu.sync_copy(ones_v, acc_sh.at[dst_v], add=True)
            return carry

        lax.fori_loop(0, NCHUNK, body, 0)
        plsc.subcore_barrier()
        pltpu.sync_copy(acc_sh.at[pl.ds(r0, RPW)], out_hbm.at[c, pl.ds(r0, RPW)])

    @functools.partial(
        pl.kernel,
        mesh=mesh,
        compiler_params=params,
        out_type=jax.ShapeDtypeStruct((NC, N_PAD, H), jnp.float32),
        scratch_types=[
            pltpu.VMEM_SHARED((N_PAD, H), jnp.float32),
            pltpu.VMEM((CHUNK,), jnp.int32),
            pltpu.VMEM((CHUNK,), jnp.int32),
            pltpu.VMEM((CHUNK, H), jnp.float32),
            pltpu.SemaphoreType.DMA,
        ],
    )
    def _sc_propagate(y_hbm, src_hbm, dst_hbm, out_hbm, acc_sh, src_v, dst_v,
                      rows_v, sem):
        c = lax.axis_index("c")
        s = lax.axis_index("s")
        w = c * NS + s
        r0 = s * RPW
        pltpu.sync_copy(y_hbm.at[pl.ds(r0, RPW)], acc_sh.at[pl.ds(r0, RPW)])
        plsc.subcore_barrier()
        base = w * EPW

        def body(j, carry):
            b = base + j * CHUNK
            pltpu.sync_copy(src_hbm.at[pl.ds(b, CHUNK)], src_v)
            pltpu.async_copy(y_hbm.at[src_v], rows_v, sem).wait()
            pltpu.sync_copy(dst_hbm.at[pl.ds(b, CHUNK)], dst_v)
            pltpu.sync_copy(rows_v, acc_sh.at[dst_v], add=True)
            return carry

        lax.fori_loop(0, NCHUNK, body, 0)
        plsc.subcore_barrier()
        pltpu.sync_copy(acc_sh.at[pl.ds(r0, RPW)], out_hbm.at[c, pl.ds(r0, RPW)])

    return _sc_degree, _sc_propagate


def _dinv(degp0, degp1):
    deg = degp0 + degp1 - 1.0
    return lax.rsqrt(jnp.maximum(deg, 1.0))


def _mm_scale_body(x_ref, w_ref, degp_ref, y_ref):
    xw = jnp.dot(x_ref[...], w_ref[...], preferred_element_type=jnp.float32)
    y_ref[...] = xw * _dinv(degp_ref[0], degp_ref[1])


def _mid_body(accp_ref, degp_ref, y1_ref, b1_ref, yh_ref):
    i = pl.program_id(0)
    dinv = _dinv(degp_ref[0], degp_ref[1])
    acc = accp_ref[0] + accp_ref[1] - y1_ref[...]
    h = jnp.maximum(dinv * acc + b1_ref[...], 0.0)
    rid = i * BLK + lax.broadcasted_iota(jnp.int32, (BLK, H), 0)
    yh_ref[...] = jnp.where(rid < N, dinv * h, 0.0)


def _final_body(accp_ref, degp_ref, yh_ref, w2t_ref, b2_ref, out_ref):
    dinv = _dinv(degp_ref[0], degp_ref[1])
    z = dinv * (accp_ref[0] + accp_ref[1] - yh_ref[...])
    w2t = w2t_ref[...]
    b2 = b2_ref[...]
    ls = [
        jnp.sum(z * w2t[c_:c_ + 1, :], axis=1, keepdims=True) + b2[0, c_]
        for c_ in range(C)
    ]
    m = jnp.maximum(jnp.maximum(ls[0], ls[1]), ls[2])
    se = sum(jnp.exp(l - m) for l in ls)
    lse = m + jnp.log(se)
    out_ref[...] = jnp.concatenate([l - lse for l in ls], axis=1)


_degp_spec = pl.BlockSpec((NC, BLK, H), lambda i: (0, i, 0))
_row_spec = pl.BlockSpec((BLK, H), lambda i: (i, 0))

_mm_scale = pl.pallas_call(
    _mm_scale_body,
    grid=(GRID,),
    in_specs=[
        pl.BlockSpec((BLK, D_PAD), lambda i: (i, 0)),
        pl.BlockSpec((D_PAD, H), lambda i: (0, 0)),
        _degp_spec,
    ],
    out_specs=_row_spec,
    out_shape=jax.ShapeDtypeStruct((N_PAD, H), jnp.float32),
)

_mid = pl.pallas_call(
    _mid_body,
    grid=(GRID,),
    in_specs=[
        _degp_spec,
        _degp_spec,
        _row_spec,
        pl.BlockSpec((1, H), lambda i: (0, 0)),
    ],
    out_specs=_row_spec,
    out_shape=jax.ShapeDtypeStruct((N_PAD, H), jnp.float32),
)

_final = pl.pallas_call(
    _final_body,
    grid=(GRID,),
    in_specs=[
        _degp_spec,
        _degp_spec,
        _row_spec,
        pl.BlockSpec((C, H), lambda i: (0, 0)),
        pl.BlockSpec((1, C), lambda i: (0, 0)),
    ],
    out_specs=pl.BlockSpec((BLK, C), lambda i: (i, 0)),
    out_shape=jax.ShapeDtypeStruct((N_PAD, C), jnp.float32),
)


def kernel(x, edge_index, W1, b1, W2, b2):
    src = edge_index[0]
    dst = edge_index[1]
    npad = E_PAD - E
    pad_ids = (N + (jnp.arange(npad, dtype=jnp.int32) % (N_PAD - N)))
    pad_ids = pad_ids.astype(jnp.int32)
    src_p = jnp.concatenate([src, pad_ids])
    dst_p = jnp.concatenate([dst, pad_ids])

    xp = jnp.zeros((N_PAD, D_PAD), jnp.float32).at[:N, :D_IN].set(x)
    w1p = jnp.zeros((D_PAD, H), jnp.float32).at[:D_IN].set(W1)
    ones = jnp.ones((N_PAD, H), jnp.float32)

    sc_degree, sc_propagate = _sc_kernels()
    degp = sc_degree(ones, dst_p)
    y1 = _mm_scale(xp, w1p, degp)
    acc1 = sc_propagate(y1, src_p, dst_p)
    yh = _mid(acc1, degp, y1, b1.reshape(1, H))
    acc2 = sc_propagate(yh, src_p, dst_p)
    out = _final(acc2, degp, yh, W2.T, b2.reshape(1, C))
    return out[:N]


# speedup vs baseline: 19.0660x; 1.5537x over previous
import functools

import jax
import jax.numpy as jnp
from jax import lax
from jax.experimental import pallas as pl
from jax.experimental.pallas import tpu as pltpu
from jax.experimental.pallas import tpu_sc as plsc

N = 10000
E = 160000
D_IN = 500
H = 16
C = 3

N_PAD = 10240
D_PAD = 512
NC = 2
NS = 16
NW = NC * NS
CHUNK = 128
NCHUNK = 40
E_PAD = NW * CHUNK * NCHUNK
EPW = CHUNK * NCHUNK
RPW = N_PAD // NS
BLK = 512
GRID = N_PAD // BLK

@functools.cache
def _sc_kernels():
    mesh = plsc.VectorSubcoreMesh(core_axis_name="c", subcore_axis_name="s")
    params = pltpu.CompilerParams(use_tc_tiling_on_sc=False)

    @functools.partial(
        pl.kernel,
        mesh=mesh,
        compiler_params=params,
        out_type=jax.ShapeDtypeStruct((NC, N_PAD, H), jnp.float32),
        scratch_types=[
            pltpu.VMEM_SHARED((N_PAD, H), jnp.float32),
            pltpu.VMEM((NCHUNK, CHUNK), jnp.int32),
            pltpu.VMEM((CHUNK, H), jnp.float32),
            pltpu.SemaphoreType.DMA,
        ],
    )
    def _sc_degree(ones_hbm, dst_hbm, out_hbm, acc_sh, didx, ones_v, sem_s):
        c = lax.axis_index("c")
        s = lax.axis_index("s")
        w = c * NS + s
        r0 = s * RPW
        pltpu.sync_copy(ones_hbm.at[pl.ds(r0, RPW)], acc_sh.at[pl.ds(r0, RPW)])
        pltpu.sync_copy(ones_hbm.at[pl.ds(0, CHUNK)], ones_v)
        pltpu.sync_copy(dst_hbm.at[pl.ds(w * NCHUNK, NCHUNK)], didx)
        plsc.subcore_barrier()

        def fire(j, carry):
            pltpu.async_copy(ones_v, acc_sh.at[didx.at[j]], sem_s, add=True)
            return carry

        lax.fori_loop(0, NCHUNK, fire, 0)

        def drain(j, carry):
            pltpu.make_async_copy(ones_v, acc_sh.at[didx.at[j]], sem_s).wait()
            return carry

        lax.fori_loop(0, NCHUNK, drain, 0)
        plsc.subcore_barrier()
        pltpu.sync_copy(acc_sh.at[pl.ds(r0, RPW)], out_hbm.at[c, pl.ds(r0, RPW)])

    @functools.partial(
        pl.kernel,
        mesh=mesh,
        compiler_params=params,
        out_type=jax.ShapeDtypeStruct((NC, N_PAD, H), jnp.float32),
        scratch_types=[
            pltpu.VMEM_SHARED((N_PAD, H), jnp.float32),
            pltpu.VMEM((NCHUNK, CHUNK), jnp.int32),
            pltpu.VMEM((NCHUNK, CHUNK), jnp.int32),
            pltpu.VMEM((NCHUNK, CHUNK, H), jnp.float32),
            pltpu.SemaphoreType.DMA,
            pltpu.SemaphoreType.DMA,
        ],
    )
    def _sc_propagate(y_hbm, src_hbm, dst_hbm, out_hbm, acc_sh, sidx, didx,
                      rows, sem_g, sem_s):
        c = lax.axis_index("c")
        s = lax.axis_index("s")
        w = c * NS + s
        r0 = s * RPW
        pltpu.sync_copy(y_hbm.at[pl.ds(r0, RPW)], acc_sh.at[pl.ds(r0, RPW)])
        pltpu.sync_copy(src_hbm.at[pl.ds(w * NCHUNK, NCHUNK)], sidx)
        pltpu.sync_copy(dst_hbm.at[pl.ds(w * NCHUNK, NCHUNK)], didx)
        plsc.subcore_barrier()

        def fire_g(j, carry):
            pltpu.async_copy(y_hbm.at[sidx.at[j]], rows.at[j], sem_g)
            return carry

        lax.fori_loop(0, NCHUNK, fire_g, 0)

        def drain_g(j, carry):
            pltpu.make_async_copy(y_hbm.at[sidx.at[j]], rows.at[j],
                                  sem_g).wait()
            return carry

        lax.fori_loop(0, NCHUNK, drain_g, 0)

        def fire_s(j, carry):
            pltpu.async_copy(rows.at[j], acc_sh.at[didx.at[j]], sem_s,
                             add=True)
            return carry

        lax.fori_loop(0, NCHUNK, fire_s, 0)

        def drain_s(j, carry):
            pltpu.make_async_copy(rows.at[j], acc_sh.at[didx.at[j]],
                                  sem_s).wait()
            return carry

        lax.fori_loop(0, NCHUNK, drain_s, 0)
        plsc.subcore_barrier()
        pltpu.sync_copy(acc_sh.at[pl.ds(r0, RPW)], out_hbm.at[c, pl.ds(r0, RPW)])

    return _sc_degree, _sc_propagate


def _dinv(degp0, degp1):
    deg = degp0 + degp1 - 1.0
    return lax.rsqrt(jnp.maximum(deg, 1.0))


def _mm_scale_body(x_ref, w_ref, degp_ref, y_ref):
    xw = jnp.dot(x_ref[...], w_ref[...], preferred_element_type=jnp.float32)
    y_ref[...] = xw * _dinv(degp_ref[0], degp_ref[1])


def _mid_body(accp_ref, degp_ref, y1_ref, b1_ref, yh_ref):
    i = pl.program_id(0)
    dinv = _dinv(degp_ref[0], degp_ref[1])
    acc = accp_ref[0] + accp_ref[1] - y1_ref[...]
    h = jnp.maximum(dinv * acc + b1_ref[...], 0.0)
    rid = i * BLK + lax.broadcasted_iota(jnp.int32, (BLK, H), 0)
    yh_ref[...] = jnp.where(rid < N, dinv * h, 0.0)


def _final_body(accp_ref, degp_ref, yh_ref, w2t_ref, b2_ref, out_ref):
    dinv = _dinv(degp_ref[0], degp_ref[1])
    z = dinv * (accp_ref[0] + accp_ref[1] - yh_ref[...])
    w2t = w2t_ref[...]
    b2 = b2_ref[...]
    ls = [
        jnp.sum(z * w2t[c_:c_ + 1, :], axis=1, keepdims=True) + b2[0, c_]
        for c_ in range(C)
    ]
    m = jnp.maximum(jnp.maximum(ls[0], ls[1]), ls[2])
    se = sum(jnp.exp(l - m) for l in ls)
    lse = m + jnp.log(se)
    out_ref[...] = jnp.concatenate([l - lse for l in ls], axis=1)


_degp_spec = pl.BlockSpec((NC, BLK, H), lambda i: (0, i, 0))
_row_spec = pl.BlockSpec((BLK, H), lambda i: (i, 0))

_mm_scale = pl.pallas_call(
    _mm_scale_body,
    grid=(GRID,),
    in_specs=[
        pl.BlockSpec((BLK, D_PAD), lambda i: (i, 0)),
        pl.BlockSpec((D_PAD, H), lambda i: (0, 0)),
        _degp_spec,
    ],
    out_specs=_row_spec,
    out_shape=jax.ShapeDtypeStruct((N_PAD, H), jnp.float32),
)

_mid = pl.pallas_call(
    _mid_body,
    grid=(GRID,),
    in_specs=[
        _degp_spec,
        _degp_spec,
        _row_spec,
        pl.BlockSpec((1, H), lambda i: (0, 0)),
    ],
    out_specs=_row_spec,
    out_shape=jax.ShapeDtypeStruct((N_PAD, H), jnp.float32),
)

_final = pl.pallas_call(
    _final_body,
    grid=(GRID,),
    in_specs=[
        _degp_spec,
        _degp_spec,
        _row_spec,
        pl.BlockSpec((C, H), lambda i: (0, 0)),
        pl.BlockSpec((1, C), lambda i: (0, 0)),
    ],
    out_specs=pl.BlockSpec((BLK, C), lambda i: (i, 0)),
    out_shape=jax.ShapeDtypeStruct((N_PAD, C), jnp.float32),
)


def kernel(x, edge_index, W1, b1, W2, b2):
    src = edge_index[0]
    dst = edge_index[1]
    npad = E_PAD - E
    pad_ids = (N + (jnp.arange(npad, dtype=jnp.int32) % (N_PAD - N)))
    pad_ids = pad_ids.astype(jnp.int32)
    src_p = jnp.concatenate([src, pad_ids]).reshape(E_PAD // CHUNK, CHUNK)
    dst_p = jnp.concatenate([dst, pad_ids]).reshape(E_PAD // CHUNK, CHUNK)

    xp = jnp.zeros((N_PAD, D_PAD), jnp.float32).at[:N, :D_IN].set(x)
    w1p = jnp.zeros((D_PAD, H), jnp.float32).at[:D_IN].set(W1)
    ones = jnp.ones((N_PAD, H), jnp.float32)

    sc_degree, sc_propagate = _sc_kernels()
    degp = sc_degree(ones, dst_p)
    y1 = _mm_scale(xp, w1p, degp)
    acc1 = sc_propagate(_sc_layout(y1), src_p, dst_p)
    yh = _mid(acc1, degp, y1, b1.reshape(1, H))
    acc2 = sc_propagate(_sc_layout(yh), src_p, dst_p)
    out = _final(acc2, degp, yh, W2.T, b2.reshape(1, C))
    return out[:N]


def _sc_layout(a):
    packed = lax.optimization_barrier(a.reshape(N_PAD * H // 128, 128))
    return packed.reshape(N_PAD, H)
